# Initial kernel scaffold; baseline (speedup 1.0000x reference)
import jax
import jax.numpy as jnp
from jax.experimental import pallas as pl


def kernel(x, table):
    raise NotImplementedError("write your pallas kernel here")


# speedup vs baseline: 1.4789x; 1.4789x over previous
import functools

import jax
import jax.numpy as jnp
from jax import lax
from jax.experimental import pallas as pl
from jax.experimental.pallas import tpu as pltpu
from jax.experimental.pallas import tpu_sc as plsc


@functools.lru_cache(maxsize=None)
def _make_gather(B: int, D: int, C: int):
    info = plsc.get_sparse_core_info()
    NC, NS = info.num_cores, info.num_subcores
    NW = NC * NS
    assert B % NW == 0
    b_per_w = B // NW
    assert b_per_w % C == 0
    n_chunks = b_per_w // C
    mesh = plsc.VectorSubcoreMesh(core_axis_name="c", subcore_axis_name="s")

    @functools.partial(
        pl.kernel,
        mesh=mesh,
        out_type=jax.ShapeDtypeStruct((B, D), jnp.float32),
        scratch_types=[
            pltpu.VMEM((C,), jnp.int32),
            pltpu.VMEM((C, D), jnp.float32),
            pltpu.SemaphoreType.DMA,
        ],
        compiler_params=pltpu.CompilerParams(use_tc_tiling_on_sc=False),
    )
    def gather_kernel(table_hbm, idx_hbm, out_hbm, idx_v, rows_v, sem):
        wid = lax.axis_index("s") * NC + lax.axis_index("c")
        base = wid * b_per_w

        def body(i, carry):
            off = base + i * C
            pltpu.sync_copy(idx_hbm.at[pl.ds(off, C)], idx_v)
            pltpu.async_copy(table_hbm.at[idx_v], rows_v, sem).wait()
            pltpu.sync_copy(rows_v, out_hbm.at[pl.ds(off, C)])
            return carry

        lax.fori_loop(0, n_chunks, body, 0)

    return gather_kernel


def kernel(x, table):
    B0, H = x.shape
    V, D = table.shape
    xf = x.reshape(B0 * H).astype(jnp.int32)
    out = _make_gather(B0 * H, D, 1600)(table, xf)
    return out.reshape(B0, H, D)


# speedup vs baseline: 1.5033x; 1.0165x over previous
import functools

import jax
import jax.numpy as jnp
from jax import lax
from jax.experimental import pallas as pl
from jax.experimental.pallas import tpu as pltpu
from jax.experimental.pallas import tpu_sc as plsc


@functools.lru_cache(maxsize=None)
def _make_gather(B: int, D: int, C: int):
    info = plsc.get_sparse_core_info()
    NC, NS = info.num_cores, info.num_subcores
    NW = NC * NS
    assert B % NW == 0
    b_per_w = B // NW
    assert b_per_w % C == 0
    n_chunks = b_per_w // C
    assert n_chunks % 2 == 0 and n_chunks >= 4
    mesh = plsc.VectorSubcoreMesh(core_axis_name="c", subcore_axis_name="s")

    @functools.partial(
        pl.kernel,
        mesh=mesh,
        out_type=jax.ShapeDtypeStruct((B, D), jnp.float32),
        scratch_types=[
            pltpu.VMEM((n_chunks, C), jnp.int32),
            pltpu.VMEM((C, D), jnp.float32),
            pltpu.VMEM((C, D), jnp.float32),
            pltpu.SemaphoreType.DMA,
            pltpu.SemaphoreType.DMA,
            pltpu.SemaphoreType.DMA,
            pltpu.SemaphoreType.DMA,
        ],
        compiler_params=pltpu.CompilerParams(use_tc_tiling_on_sc=False),
    )
    def gather_kernel(table_hbm, idx_hbm, out_hbm, idx_v, r0, r1, gs0, gs1,
                      ss0, ss1):
        wid = lax.axis_index("s") * NC + lax.axis_index("c")
        base = wid * b_per_w
        pltpu.sync_copy(idx_hbm.at[wid], idx_v)

        def g_start(g, r, s):
            pltpu.async_copy(table_hbm.at[idx_v.at[g]], r, s)

        def g_wait(g, r, s):
            pltpu.make_async_copy(table_hbm.at[idx_v.at[g]], r, s).wait()

        def s_start(g, r, s):
            pltpu.async_copy(r, out_hbm.at[pl.ds(base + g * C, C)], s)

        def s_wait(g, r, s):
            pltpu.make_async_copy(r, out_hbm.at[pl.ds(base + g * C, C)],
                                  s).wait()

        g_start(0, r0, gs0)
        g_start(1, r1, gs1)
        g_wait(0, r0, gs0)
        s_start(0, r0, ss0)

        def body(io, carry):
            g = 1 + 2 * io
            s_wait(g - 1, r0, ss0)
            g_start(g + 1, r0, gs0)
            g_wait(g, r1, gs1)
            s_start(g, r1, ss1)
            s_wait(g, r1, ss1)
            g_start(g + 2, r1, gs1)
            g_wait(g + 1, r0, gs0)
            s_start(g + 1, r0, ss0)
            return carry

        lax.fori_loop(0, (n_chunks - 2) // 2, body, 0)

        g = n_chunks - 1
        g_wait(g, r1, gs1)
        s_start(g, r1, ss1)
        s_wait(g - 1, r0, ss0)
        s_wait(g, r1, ss1)

    return gather_kernel


def kernel(x, table):
    B0, H = x.shape
    V, D = table.shape
    B = B0 * H
    C = 1600
    info = plsc.get_sparse_core_info()
    NW = info.num_cores * info.num_subcores
    n_chunks = B // NW // C
    xf = x.reshape(NW, n_chunks, C).astype(jnp.int32)
    out = _make_gather(B, D, C)(table, xf)
    return out.reshape(B0, H, D)
